# baseline (speedup 1.0000x reference)
import functools

import jax
import jax.numpy as jnp
from jax.experimental import pallas as pl
from jax.experimental.pallas import tpu as pltpu


def _body(adj_ref, ids_ref, W1_ref, b1_ref, W2_ref, b2_ref, out_ref,
          carry_ref, a_ref, *, T, NC):
    j = pl.program_id(0)

    @pl.when(j == 0)
    def _():
        carry_ref[...] = jnp.zeros_like(carry_ref)

    H1 = jnp.dot(ids_ref[...], W1_ref[...], preferred_element_type=jnp.float32)
    H2 = jnp.dot(H1, W2_ref[...], preferred_element_type=jnp.float32)
    c = jnp.dot(b1_ref[...], W2_ref[...], preferred_element_type=jnp.float32)

    Qq = ids_ref.shape[0]
    Dd = ids_ref.shape[-1]
    Bb = adj_ref.shape[0]

    row = jax.lax.broadcasted_iota(jnp.int32, (1, Qq, Qq), 1)
    col = jax.lax.broadcasted_iota(jnp.int32, (1, Qq, Qq), 2)
    eye = row == col

    carry = carry_ref[...]
    for t in range(T - 1, -1, -1):
        carry = 0.5 * (carry + adj_ref[:, t])
        a_ref[:, t] = jnp.where(eye, 1.0, carry)
    carry_ref[...] = carry

    a_flat = a_ref[...].reshape(Bb * T * Qq, Qq)

    ones8 = jnp.ones((Qq, 8), jnp.float32)
    deg = jnp.dot(a_flat, ones8, preferred_element_type=jnp.float32)
    dis = jax.lax.rsqrt(deg)
    dis_row = jnp.swapaxes(
        dis[:, :1].reshape(Bb * T, Qq, 1), 1, 2)

    an_flat = (a_flat * dis[:, :1]).reshape(Bb * T, Qq, Qq) * dis_row
    an_flat = an_flat.reshape(Bb * T * Qq, Qq)

    rhs = jnp.concatenate(
        [H2, jnp.ones((Qq, 64), jnp.float32)], axis=1)
    m = jnp.dot(an_flat, rhs, preferred_element_type=jnp.float32)
    y = m[:, :Dd].reshape(Bb * T, Qq, Dd)
    rs = m[:, Dd:Dd + 1]

    z = jax.lax.dot_general(
        an_flat.reshape(Bb * T, Qq, Qq), y,
        dimension_numbers=(((2,), (1,)), ((0,), (0,))),
        preferred_element_type=jnp.float32)

    out = (z.reshape(Bb * T * Qq, Dd)
           + rs * c.reshape(1, Dd)
           + b2_ref[...].reshape(1, Dd))
    out_ref[...] = out.reshape(Bb, T, Qq, Dd)


def kernel(adj_matrices, ids, W1, b1, W2, b2):
    B, S, Q, _ = adj_matrices.shape
    D = ids.shape[-1]
    T = 16
    NC = S // T

    b1r = b1.reshape(1, D)
    b2r = b2.reshape(1, D)

    body = functools.partial(_body, T=T, NC=NC)
    out = pl.pallas_call(
        body,
        grid=(NC,),
        in_specs=[
            pl.BlockSpec((B, T, Q, Q), lambda j: (0, NC - 1 - j, 0, 0)),
            pl.BlockSpec((Q, D), lambda j: (0, 0)),
            pl.BlockSpec((D, D), lambda j: (0, 0)),
            pl.BlockSpec((1, D), lambda j: (0, 0)),
            pl.BlockSpec((D, D), lambda j: (0, 0)),
            pl.BlockSpec((1, D), lambda j: (0, 0)),
        ],
        out_specs=pl.BlockSpec((B, T, Q, D), lambda j: (0, NC - 1 - j, 0, 0)),
        out_shape=jax.ShapeDtypeStruct((B, S, Q, D), jnp.float32),
        scratch_shapes=[
            pltpu.VMEM((B, Q, Q), jnp.float32),
            pltpu.VMEM((B, T, Q, Q), jnp.float32),
        ],
        compiler_params=pltpu.CompilerParams(
            dimension_semantics=("arbitrary",),
        ),
    )(adj_matrices, ids, W1, b1r, W2, b2r)
    return out


# speedup vs baseline: 1.9983x; 1.9983x over previous
import functools

import jax
import jax.numpy as jnp
from jax.experimental import pallas as pl
from jax.experimental.pallas import tpu as pltpu


def _fast_rsqrt(x):
    i = jax.lax.bitcast_convert_type(x, jnp.int32)
    i = jnp.int32(0x5F3759DF) - jax.lax.shift_right_logical(i, 1)
    y = jax.lax.bitcast_convert_type(i, jnp.float32)
    h = 0.5 * x
    y = y * (1.5 - h * y * y)
    y = y * (1.5 - h * y * y)
    y = y * (1.5 - h * y * y)
    return y


def _body(adj_ref, ids_ref, W1_ref, b1_ref, W2_ref, b2_ref, out_ref,
          carry_ref, m_ref, d_ref, ne_ref, eye_ref, *, TS, HF, NC):
    b = pl.program_id(0)
    j = pl.program_id(1)

    Qq = ids_ref.shape[0]
    Dd = ids_ref.shape[-1]

    @pl.when((b == 0) & (j == 0))
    def _():
        k = jax.lax.broadcasted_iota(jnp.int32, (TS, TS), 0)
        s = jax.lax.broadcasted_iota(jnp.int32, (TS, TS), 1)
        dec = jnp.exp2(-(k - s + 1).astype(jnp.float32))
        m_ref[...] = jnp.where(k >= s, dec, 0.0)
        s8 = jax.lax.broadcasted_iota(jnp.int32, (8, TS), 1)
        d_ref[...] = jnp.exp2((s8 - TS).astype(jnp.float32))
        ri = jax.lax.broadcasted_iota(jnp.int32, (Qq, Qq, HF), 0)
        rj = jax.lax.broadcasted_iota(jnp.int32, (Qq, Qq, HF), 1)
        eq = ri == rj
        ne_ref[...] = jnp.where(eq, 0.0, 1.0).reshape(Qq * Qq, HF)
        eye_ref[...] = jnp.where(eq, 1.0, 0.0).reshape(Qq * Qq, HF)

    @pl.when(j == 0)
    def _():
        carry_ref[...] = jnp.zeros_like(carry_ref)

    H1 = jnp.dot(ids_ref[...], W1_ref[...], preferred_element_type=jnp.float32)
    H2 = jnp.dot(H1, W2_ref[...], preferred_element_type=jnp.float32)
    c = jnp.dot(b1_ref[...], W2_ref[...], preferred_element_type=jnp.float32)

    af = adj_ref[0].reshape(Qq * Qq, TS)
    w = (jnp.dot(af, m_ref[...], preferred_element_type=jnp.float32)
         + carry_ref[...] * d_ref[0:1, :])
    carry_ref[...] = jnp.broadcast_to(w[:, 0:1], (Qq * Qq, TS))

    ne = ne_ref[...]
    eye = eye_ref[...]
    for h in range(TS // HF):
        wh = w[:, h * HF:(h + 1) * HF]
        t = wh * ne
        deg = jnp.sum(t.reshape(Qq, Qq, HF), axis=1) + 1.0
        dis = _fast_rsqrt(deg)
        a = t + eye
        an = a.reshape(Qq, Qq, HF) * dis[:, None, :] * dis[None, :, :]

        an_t = jnp.transpose(an, (2, 0, 1))
        anf = an_t.reshape(HF * Qq, Qq)
        y = jnp.dot(anf, H2, preferred_element_type=jnp.float32) + c
        z = jax.lax.dot_general(
            an_t, y.reshape(HF, Qq, Dd),
            dimension_numbers=(((2,), (1,)), ((0,), (0,))),
            preferred_element_type=jnp.float32)
        o = z + b2_ref[...].reshape(1, 1, Dd)
        out_ref[0, :, :, h * HF:(h + 1) * HF] = jnp.transpose(o, (1, 2, 0))


def kernel(adj_matrices, ids, W1, b1, W2, b2):
    B, S, Q, _ = adj_matrices.shape
    D = ids.shape[-1]
    TS = 256
    HF = 128
    NC = S // TS

    adj_t = jnp.transpose(adj_matrices, (0, 2, 3, 1))
    b1r = b1.reshape(1, D)
    b2r = b2.reshape(1, D)

    body = functools.partial(_body, TS=TS, HF=HF, NC=NC)
    out_t = pl.pallas_call(
        body,
        grid=(B, NC),
        in_specs=[
            pl.BlockSpec((1, Q, Q, TS), lambda b, j: (b, 0, 0, NC - 1 - j)),
            pl.BlockSpec((Q, D), lambda b, j: (0, 0)),
            pl.BlockSpec((D, D), lambda b, j: (0, 0)),
            pl.BlockSpec((1, D), lambda b, j: (0, 0)),
            pl.BlockSpec((D, D), lambda b, j: (0, 0)),
            pl.BlockSpec((1, D), lambda b, j: (0, 0)),
        ],
        out_specs=pl.BlockSpec((1, Q, D, TS), lambda b, j: (b, 0, 0, NC - 1 - j)),
        out_shape=jax.ShapeDtypeStruct((B, Q, D, S), jnp.float32),
        scratch_shapes=[
            pltpu.VMEM((Q * Q, TS), jnp.float32),
            pltpu.VMEM((TS, TS), jnp.float32),
            pltpu.VMEM((8, TS), jnp.float32),
            pltpu.VMEM((Q * Q, HF), jnp.float32),
            pltpu.VMEM((Q * Q, HF), jnp.float32),
        ],
        compiler_params=pltpu.CompilerParams(
            dimension_semantics=("arbitrary", "arbitrary"),
        ),
    )(adj_t, ids, W1, b1r, W2, b2r)
    return jnp.transpose(out_t, (0, 3, 1, 2))
